# baseline (speedup 1.0000x reference)
import functools
import math

import jax
import jax.numpy as jnp
from jax import lax
from jax.experimental import pallas as pl
from jax.experimental.pallas import tpu as pltpu
from jax.experimental.pallas import tpu_sc as plsc

N, T, C, H2, K = 10000, 4, 128, 64, 8
TC_FLAT = T * C
N_PAD = 10240
RB = 256
CT = 2048
_INTERPRET = False


def _dense_body(x_ref, wmsg_ref, bmsg_ref, wself_ref, bself_ref,
                wproj_ref, sv_ref, sb_ref,
                msg_ref, selfp_ref, s_ref, e_ref, et_ref):
    xb = x_ref[...]
    msg = jax.lax.dot_general(
        xb, wmsg_ref[...], (((1,), (0,)), ((), ())),
        preferred_element_type=jnp.float32) + bmsg_ref[...]
    msg_ref[...] = msg
    sp = jax.lax.dot_general(
        xb, wself_ref[...], (((1,), (0,)), ((), ())),
        preferred_element_type=jnp.float32) + bself_ref[...]
    selfp_ref[...] = sp.reshape(RB, T, C).transpose(1, 0, 2)
    s = jax.lax.dot_general(
        msg, sv_ref[...], (((1,), (0,)), ((), ())),
        preferred_element_type=jnp.float32) + sb_ref[...]
    s_ref[...] = s.T
    e_un = jax.lax.dot_general(
        xb, wproj_ref[...], (((1,), (0,)), ((), ())),
        preferred_element_type=jnp.float32)
    nrm = jnp.sqrt(jnp.sum(e_un * e_un, axis=1, keepdims=True))
    e = e_un / (nrm + 1e-12)
    e_ref[...] = e
    et_ref[...] = e.T


def _dense_call(x_flat, wmsg, bmsg, wself, bself, wproj, sv, sb):
    grid = (N_PAD // RB,)
    return pl.pallas_call(
        _dense_body,
        grid=grid,
        in_specs=[
            pl.BlockSpec((RB, TC_FLAT), lambda i: (i, 0)),
            pl.BlockSpec((TC_FLAT, TC_FLAT), lambda i: (0, 0)),
            pl.BlockSpec((1, TC_FLAT), lambda i: (0, 0)),
            pl.BlockSpec((TC_FLAT, TC_FLAT), lambda i: (0, 0)),
            pl.BlockSpec((1, TC_FLAT), lambda i: (0, 0)),
            pl.BlockSpec((TC_FLAT, H2), lambda i: (0, 0)),
            pl.BlockSpec((TC_FLAT, 8), lambda i: (0, 0)),
            pl.BlockSpec((1, 8), lambda i: (0, 0)),
        ],
        out_specs=[
            pl.BlockSpec((RB, TC_FLAT), lambda i: (i, 0)),
            pl.BlockSpec((T, RB, C), lambda i: (0, i, 0)),
            pl.BlockSpec((8, RB), lambda i: (0, i)),
            pl.BlockSpec((RB, H2), lambda i: (i, 0)),
            pl.BlockSpec((H2, RB), lambda i: (0, i)),
        ],
        out_shape=[
            jax.ShapeDtypeStruct((N_PAD, TC_FLAT), jnp.float32),
            jax.ShapeDtypeStruct((T, N_PAD, C), jnp.float32),
            jax.ShapeDtypeStruct((8, N_PAD), jnp.float32),
            jax.ShapeDtypeStruct((N_PAD, H2), jnp.float32),
            jax.ShapeDtypeStruct((H2, N_PAD), jnp.float32),
        ],
        interpret=_INTERPRET,
    )(x_flat, wmsg, bmsg, wself, bself, wproj, sv, sb)


def _topk_body(e_ref, et_ref, tv_ref, ti_ref):
    i = pl.program_id(0)
    er = e_ref[...]
    row_g = i * RB + jax.lax.broadcasted_iota(jnp.int32, (RB, 1), 0)
    run_v = jnp.full((RB, K), -jnp.inf, jnp.float32)
    run_i = jnp.zeros((RB, K), jnp.int32)
    pos16 = jax.lax.broadcasted_iota(jnp.int32, (RB, 2 * K), 1)
    for ct in range(N_PAD // CT):
        sim = jax.lax.dot_general(
            er, et_ref[:, ct * CT:(ct + 1) * CT], (((1,), (0,)), ((), ())),
            preferred_element_type=jnp.float32)
        colg = ct * CT + jax.lax.broadcasted_iota(jnp.int32, (RB, CT), 1)
        sim = jnp.where((colg == row_g) | (colg >= N), -jnp.inf, sim)
        tvals, tidx = [], []
        for _ in range(K):
            m = jnp.max(sim, axis=1, keepdims=True)
            cand = jnp.where(sim == m, colg, jnp.int32(2 ** 30))
            am = jnp.min(cand, axis=1, keepdims=True)
            sim = jnp.where(colg == am, -jnp.inf, sim)
            tvals.append(m)
            tidx.append(am)
        cv = jnp.concatenate([run_v] + tvals, axis=1)
        ci = jnp.concatenate([run_i] + tidx, axis=1)
        nv, ni = [], []
        for _ in range(K):
            m = jnp.max(cv, axis=1, keepdims=True)
            p = jnp.where(cv == m, pos16, jnp.int32(2 ** 30))
            pm = jnp.min(p, axis=1, keepdims=True)
            sel = pos16 == pm
            ni.append(jnp.max(jnp.where(sel, ci, -1), axis=1, keepdims=True))
            nv.append(m)
            cv = jnp.where(sel, -jnp.inf, cv)
        run_v = jnp.concatenate(nv, axis=1)
        run_i = jnp.concatenate(ni, axis=1)
    tv_ref[...] = run_v
    ti_ref[...] = run_i


def _topk_call(e, et):
    grid = (N_PAD // RB,)
    return pl.pallas_call(
        _topk_body,
        grid=grid,
        in_specs=[
            pl.BlockSpec((RB, H2), lambda i: (i, 0)),
            pl.BlockSpec((H2, N_PAD), lambda i: (0, 0)),
        ],
        out_specs=[
            pl.BlockSpec((RB, K), lambda i: (i, 0)),
            pl.BlockSpec((RB, K), lambda i: (i, 0)),
        ],
        out_shape=[
            jax.ShapeDtypeStruct((N_PAD, K), jnp.float32),
            jax.ShapeDtypeStruct((N_PAD, K), jnp.int32),
        ],
        interpret=_INTERPRET,
    )(e, et)


EP = 245760
E_ROWS = EP // 128
E_CH_ROWS = E_ROWS // 16
BLK_ROWS = 1
BLK = BLK_ROWS * 128
NBLK = E_CH_ROWS // BLK_ROWS
NPT = N_PAD // 16
FCH = 32


def _edge_call(msg4, sT, selfp_t, src2, dst2, ea2, zrow, zd):
    mesh = plsc.VectorSubcoreMesh(core_axis_name="c", subcore_axis_name="s")

    @functools.partial(
        pl.kernel,
        mesh=mesh,
        compiler_params=pltpu.CompilerParams(needs_layout_passes=False),
        out_type=jax.ShapeDtypeStruct((T, N_PAD, C), jnp.float32),
        scratch_types=[
            pltpu.VMEM_SHARED((N_PAD, C), jnp.float32),
            pltpu.VMEM_SHARED((N_PAD,), jnp.float32),
            pltpu.VMEM((BLK, C), jnp.float32),
            pltpu.VMEM((N_PAD,), jnp.float32),
            pltpu.VMEM((N_PAD,), jnp.float32),
            pltpu.VMEM((BLK_ROWS, 128), jnp.int32),
            pltpu.VMEM((BLK_ROWS, 128), jnp.int32),
            pltpu.VMEM((BLK_ROWS, 128), jnp.float32),
            pltpu.VMEM((BLK_ROWS, 128), jnp.float32),
            pltpu.VMEM((BLK,), jnp.float32),
            pltpu.VMEM((BLK_ROWS, 128), jnp.int32),
            pltpu.VMEM((FCH,), jnp.float32),
            pltpu.SemaphoreType.DMA,
        ],
    )
    def k(msg4_h, sT_h, selfp_h, src_h, dst_h, ea_h, zrow_h, zd_h, out_h,
          agg_sh, den_sh, rows, ssrc_t, sdst_t, src_b, dst_b, ea_b, ex_b,
          wt_b, mix_b, inv_b, gsem):
        cc = lax.axis_index("c")
        ss = lax.axis_index("s")
        iota16 = lax.broadcasted_iota(jnp.int32, (16,), 0)
        z16 = jnp.zeros((16,), jnp.int32)
        n0 = ss * NPT
        r_base = ss * E_CH_ROWS
        for tp in range(2):
            t = 2 * cc + tp
            pltpu.sync_copy(zrow_h, agg_sh.at[pl.ds(n0, NPT)])
            pltpu.sync_copy(zd_h, den_sh.at[pl.ds(n0, NPT)])
            pltpu.sync_copy(sT_h.at[t], ssrc_t)
            pltpu.sync_copy(sT_h.at[t + T], sdst_t)
            plsc.subcore_barrier()

            def blk_body(b, carry):
                r0 = r_base + b * BLK_ROWS
                pltpu.sync_copy(src_h.at[pl.ds(r0, BLK_ROWS)], src_b)
                pltpu.sync_copy(dst_h.at[pl.ds(r0, BLK_ROWS)], dst_b)
                pltpu.sync_copy(ea_h.at[pl.ds(r0, BLK_ROWS)], ea_b)
                for q in range(BLK_ROWS):
                    for j in range(128 // 16):
                        sl = pl.ds(j * 16, 16)
                        sv = src_b[q, sl]
                        dv = dst_b[q, sl]
                        a = (plsc.load_gather(ssrc_t, [sv])
                             + plsc.load_gather(sdst_t, [dv]))
                        a = jnp.maximum(a, 0.2 * a)
                        ex = jnp.exp(a)
                        ex_b[q, sl] = ex
                        wt_b[pl.ds(q * 128 + j * 16, 16)] = ex * ea_b[q, sl]
                        mix_b[q, sl] = sv * T + t
                for q in range(BLK_ROWS):
                    pltpu.sync_copy(ex_b.at[q], den_sh.at[dst_b.at[q]],
                                    add=True)
                cps = [pltpu.async_copy(msg4_h.at[mix_b.at[q]],
                                        rows.at[pl.ds(q * 128, 128)], gsem)
                       for q in range(BLK_ROWS)]
                for cp in cps:
                    cp.wait()

                def e_body(ie, c2):
                    w16 = plsc.load_gather(wt_b, [z16 + ie])
                    rid = z16 + ie
                    for kk in range(C // 16):
                        col = iota16 + (kk * 16)
                        v = plsc.load_gather(rows, [rid, col])
                        plsc.store_scatter(rows, [rid, col], v * w16)
                    return c2
                lax.fori_loop(0, BLK, e_body, 0)
                for q in range(BLK_ROWS):
                    pltpu.sync_copy(rows.at[pl.ds(q * 128, 128)],
                                    agg_sh.at[dst_b.at[q]], add=True)
                return carry

            lax.fori_loop(0, NBLK, blk_body, 0)
            plsc.subcore_barrier()
            for ck in range(NPT // FCH):
                nb = n0 + ck * FCH
                pltpu.sync_copy(den_sh.at[pl.ds(nb, FCH)], inv_b)
                for j in range(FCH // 16):
                    sl = pl.ds(j * 16, 16)
                    inv_b[sl] = 1.0 / (inv_b[sl] + 1e-16)
                pltpu.sync_copy(agg_sh.at[pl.ds(nb, FCH)],
                                rows.at[pl.ds(0, FCH)])
                pltpu.sync_copy(selfp_h.at[t, pl.ds(nb, FCH)],
                                rows.at[pl.ds(FCH, FCH)])

                def n_body(jn, c2):
                    w16 = plsc.load_gather(inv_b, [z16 + jn])
                    for kk in range(C // 16):
                        col = iota16 + (kk * 16)
                        v = plsc.load_gather(rows, [z16 + jn, col])
                        sv_ = plsc.load_gather(rows, [z16 + (FCH + jn), col])
                        plsc.store_scatter(rows, [z16 + (2 * FCH + jn), col],
                                           v * w16 + sv_)
                    return c2
                lax.fori_loop(0, FCH, n_body, 0)
                pltpu.sync_copy(rows.at[pl.ds(2 * FCH, FCH)],
                                out_h.at[t, pl.ds(nb, FCH)])
            plsc.subcore_barrier()

    return k(msg4, sT, selfp_t, src2, dst2, ea2, zrow, zd)


def kernel(x, edge_index, edge_attr, node_proj_W, mix_logit, conv_W, conv_b,
           lin_msg_W, lin_msg_b, lin_self_W, lin_self_b,
           att_src_W, att_src_b, att_dst_W, att_dst_b):
    f32 = jnp.float32
    x_flat = x.reshape(N, TC_FLAT)
    x_flat = jnp.pad(x_flat, ((0, N_PAD - N), (0, 0)))

    eyeT = jnp.eye(T, dtype=f32)
    blocks = []
    for t_in in range(T):
        row = []
        for t_out in range(T):
            k = t_in - t_out + 1
            if 0 <= k <= 2:
                row.append(conv_W[:, :, k].T)
            else:
                row.append(jnp.zeros((C, C), f32))
        blocks.append(jnp.concatenate(row, axis=1))
    wconv = jnp.concatenate(blocks, axis=0)
    wm_bd = jnp.kron(eyeT, lin_msg_W.T)
    wmsg = wconv @ wm_bd
    bmsg_t = conv_b @ lin_msg_W.T + lin_msg_b
    bmsg = jnp.tile(bmsg_t, (T,))[None, :]
    wself = jnp.kron(eyeT, lin_self_W.T)
    bself = jnp.tile(lin_self_b, (T,))[None, :]
    wproj = jnp.tile(node_proj_W.T, (T, 1)) / T
    sv = jnp.zeros((TC_FLAT, 2 * T), f32)
    for t in range(T):
        sv = sv.at[t * C:(t + 1) * C, t].set(att_src_W[0])
        sv = sv.at[t * C:(t + 1) * C, T + t].set(att_dst_W[0])
    sb = jnp.concatenate([jnp.tile(att_src_b, (T,)),
                          jnp.tile(att_dst_b, (T,))])[None, :]

    msg_flat, selfp_t, sT, e, et = _dense_call(
        x_flat, wmsg, bmsg, wself, bself, wproj, sv, sb)
    tv_p, ti_p = _topk_call(e, et)
    tv = tv_p[:N]
    ti = ti_p[:N]

    alpha = jax.nn.sigmoid(mix_logit)
    e_fixed = edge_index.shape[1]
    n_pad_e = EP - (e_fixed + N * K)
    src_dyn = jnp.arange(N * K, dtype=jnp.int32) // K
    dst_dyn = ti.reshape(-1)
    pad_dst = N + (jnp.arange(n_pad_e, dtype=jnp.int32) % (N_PAD - N))
    srcs = jnp.concatenate([edge_index[0].astype(jnp.int32), src_dyn,
                            jnp.zeros((n_pad_e,), jnp.int32)])
    dsts = jnp.concatenate([edge_index[1].astype(jnp.int32), dst_dyn, pad_dst])
    eav = jnp.concatenate([edge_attr[:, 0] * (1.0 - alpha),
                           tv.reshape(-1) * alpha,
                           jnp.zeros((n_pad_e,), f32)])
    msg4 = msg_flat.reshape(N_PAD * T, C)
    zrow = jnp.zeros((NPT, C), f32)
    zd = jnp.zeros((NPT,), f32)
    out_t = _edge_call(msg4, sT, selfp_t,
                       srcs.reshape(E_ROWS, 128), dsts.reshape(E_ROWS, 128),
                       eav.reshape(E_ROWS, 128), zrow, zd)
    return out_t.transpose(1, 0, 2)[:N]
